# baseline (speedup 1.0000x reference)
import dataclasses
import functools

import jax
import jax.numpy as jnp
from jax import lax
from jax.experimental import pallas as pl
from jax.experimental.pallas import tpu as pltpu
from jax.experimental.pallas import tpu_sc as plsc

E_DIM = 64
R_DIM = 32
LANES = 16


def _score_kernel(num_workers, per_w, chunk, nchunks,
                  head_hbm, rel_hbm, tail_hbm, ent_hbm, reltab_hbm, trans_hbm,
                  out_hbm, hidx, ridx, tidx, hrows, trows, relemb, mat, outbuf,
                  sem):
    cid = lax.axis_index("core")
    sid = lax.axis_index("subcore")
    num_cores = 2
    wid = sid * num_cores + cid
    base = wid * per_w

    pltpu.sync_copy(head_hbm.at[pl.ds(base, per_w)], hidx)
    pltpu.sync_copy(tail_hbm.at[pl.ds(base, per_w)], tidx)
    pltpu.sync_copy(rel_hbm.at[pl.ds(base, per_w)], ridx)

    @pl.loop(0, nchunks)
    def _chunk(ci):
        off = ci * chunk
        hvec = hidx[pl.ds(off, chunk)]
        tvec = tidx[pl.ds(off, chunk)]
        rvec = ridx[pl.ds(off, chunk)]
        c1 = pltpu.async_copy(ent_hbm.at[hvec], hrows, sem)
        c2 = pltpu.async_copy(ent_hbm.at[tvec], trows, sem)
        c3 = pltpu.async_copy(reltab_hbm.at[rvec], relemb, sem)
        c4 = pltpu.async_copy(trans_hbm.at[rvec], mat, sem)
        c1.wait()
        c2.wait()
        c3.wait()
        c4.wait()

        def _sample(s, score_vec):
            acc0 = relemb[s, pl.ds(0, LANES)]
            acc1 = relemb[s, pl.ds(LANES, LANES)]
            for kk in range(E_DIM // LANES):
                d = (hrows[s, pl.ds(kk * LANES, LANES)]
                     - trows[s, pl.ds(kk * LANES, LANES)])
                for j in range(LANES):
                    k = kk * LANES + j
                    dk = d[j]
                    acc0 = acc0 + dk * mat[s, pl.ds(k * R_DIM, LANES)]
                    acc1 = acc1 + dk * mat[s, pl.ds(k * R_DIM + LANES, LANES)]
            score = jnp.sum(jnp.abs(acc0) + jnp.abs(acc1))
            lane = lax.iota(jnp.int32, LANES)
            return jnp.where(lane == s, score, score_vec)

        score_vec = lax.fori_loop(0, chunk, _sample,
                                  jnp.zeros((LANES,), jnp.float32))
        outbuf[pl.ds(off, chunk)] = score_vec

    pltpu.sync_copy(outbuf, out_hbm.at[pl.ds(base, per_w)])


def kernel(head, relation, tail, entity_table, relation_table, transfer_table):
    batch = head.shape[0]
    num_workers = 32
    per_w = batch // num_workers
    chunk = LANES
    nchunks = per_w // chunk

    head = head.astype(jnp.int32)
    relation = relation.astype(jnp.int32)
    tail = tail.astype(jnp.int32)

    mesh = plsc.VectorSubcoreMesh(core_axis_name="core", subcore_axis_name="subcore")
    body = functools.partial(_score_kernel, num_workers, per_w, chunk, nchunks)
    cp = pltpu.CompilerParams()
    fields = pltpu.CompilerParams.__dataclass_fields__
    if "needs_layout_passes" in fields:
        cp = dataclasses.replace(cp, needs_layout_passes=False)
    if "use_tc_tiling_on_sc" in fields:
        cp = dataclasses.replace(cp, use_tc_tiling_on_sc=False)
    run = pl.kernel(
        body,
        out_type=jax.ShapeDtypeStruct((batch,), jnp.float32),
        mesh=mesh,
        compiler_params=cp,
        scratch_types=[
            pltpu.VMEM((per_w,), jnp.int32),
            pltpu.VMEM((per_w,), jnp.int32),
            pltpu.VMEM((per_w,), jnp.int32),
            pltpu.VMEM((chunk, E_DIM), jnp.float32),
            pltpu.VMEM((chunk, E_DIM), jnp.float32),
            pltpu.VMEM((chunk, R_DIM), jnp.float32),
            pltpu.VMEM((chunk, E_DIM * R_DIM), jnp.float32),
            pltpu.VMEM((per_w,), jnp.float32),
            pltpu.SemaphoreType.DMA,
        ],
    )
    return run(head, relation, tail, entity_table, relation_table,
               transfer_table)


# speedup vs baseline: 1.0313x; 1.0313x over previous
import dataclasses
import functools

import jax
import jax.numpy as jnp
from jax import lax
from jax.experimental import pallas as pl
from jax.experimental.pallas import tpu as pltpu
from jax.experimental.pallas import tpu_sc as plsc

E_DIM = 64
R_DIM = 32
LANES = 16
NCHAIN = 4


def _score_kernel(per_w, chunk, nchunks,
                  head_hbm, rel_hbm, tail_hbm, ent_hbm, reltab_hbm, trans_hbm,
                  out_hbm, hidx, ridx, tidx,
                  hrows0, trows0, relemb0, mat0,
                  hrows1, trows1, relemb1, mat1,
                  outbuf, sem0, sem1):
    cid = lax.axis_index("core")
    sid = lax.axis_index("subcore")
    wid = sid * 2 + cid
    base = wid * per_w

    pltpu.sync_copy(head_hbm.at[pl.ds(base, per_w)], hidx)
    pltpu.sync_copy(tail_hbm.at[pl.ds(base, per_w)], tidx)
    pltpu.sync_copy(rel_hbm.at[pl.ds(base, per_w)], ridx)

    bufs = ((hrows0, trows0, relemb0, mat0, sem0),
            (hrows1, trows1, relemb1, mat1, sem1))

    def issue(ci, b):
        hr, tr, re, mt, sem = b
        off = ci * chunk
        hvec = hidx[pl.ds(off, chunk)]
        tvec = tidx[pl.ds(off, chunk)]
        rvec = ridx[pl.ds(off, chunk)]
        pltpu.async_copy(ent_hbm.at[hvec], hr, sem)
        pltpu.async_copy(ent_hbm.at[tvec], tr, sem)
        pltpu.async_copy(reltab_hbm.at[rvec], re, sem)
        pltpu.async_copy(trans_hbm.at[rvec], mt, sem)

    def wait(b):
        hr, tr, re, mt, sem = b
        pltpu.make_async_copy(ent_hbm.at[pl.ds(0, chunk)], hr, sem).wait()
        pltpu.make_async_copy(ent_hbm.at[pl.ds(0, chunk)], tr, sem).wait()
        pltpu.make_async_copy(reltab_hbm.at[pl.ds(0, chunk)], re, sem).wait()
        pltpu.make_async_copy(trans_hbm.at[pl.ds(0, chunk)], mt, sem).wait()

    def compute(ci, b):
        hr, tr, re, mt, _ = b
        off = ci * chunk

        def _sample(s, score_vec):
            zero = jnp.zeros((LANES,), jnp.float32)
            acc0 = [re[s, pl.ds(0, LANES)]] + [zero] * (NCHAIN - 1)
            acc1 = [re[s, pl.ds(LANES, LANES)]] + [zero] * (NCHAIN - 1)
            for kk in range(E_DIM // LANES):
                d = (hr[s, pl.ds(kk * LANES, LANES)]
                     - tr[s, pl.ds(kk * LANES, LANES)])
                for j in range(LANES):
                    k = kk * LANES + j
                    c = k % NCHAIN
                    dk = d[j]
                    acc0[c] = acc0[c] + dk * mt[s, pl.ds(k * R_DIM, LANES)]
                    acc1[c] = acc1[c] + dk * mt[s, pl.ds(k * R_DIM + LANES,
                                                         LANES)]
            t0 = (acc0[0] + acc0[1]) + (acc0[2] + acc0[3])
            t1 = (acc1[0] + acc1[1]) + (acc1[2] + acc1[3])
            score = jnp.sum(jnp.abs(t0) + jnp.abs(t1))
            lane = lax.iota(jnp.int32, LANES)
            return jnp.where(lane == s, score, score_vec)

        score_vec = lax.fori_loop(0, chunk, _sample,
                                  jnp.zeros((LANES,), jnp.float32))
        outbuf[pl.ds(off, chunk)] = score_vec

    issue(0, bufs[0])

    @pl.loop(0, nchunks, step=2)
    def _pair(ci):
        issue(ci + 1, bufs[1])
        wait(bufs[0])
        compute(ci, bufs[0])

        @pl.when(ci + 2 < nchunks)
        def _():
            issue(ci + 2, bufs[0])

        wait(bufs[1])
        compute(ci + 1, bufs[1])

    pltpu.sync_copy(outbuf, out_hbm.at[pl.ds(base, per_w)])


def kernel(head, relation, tail, entity_table, relation_table, transfer_table):
    batch = head.shape[0]
    num_workers = 32
    per_w = batch // num_workers
    chunk = LANES
    nchunks = per_w // chunk

    head = head.astype(jnp.int32)
    relation = relation.astype(jnp.int32)
    tail = tail.astype(jnp.int32)

    mesh = plsc.VectorSubcoreMesh(core_axis_name="core", subcore_axis_name="subcore")
    body = functools.partial(_score_kernel, per_w, chunk, nchunks)
    cp = pltpu.CompilerParams()
    fields = pltpu.CompilerParams.__dataclass_fields__
    if "needs_layout_passes" in fields:
        cp = dataclasses.replace(cp, needs_layout_passes=False)
    if "use_tc_tiling_on_sc" in fields:
        cp = dataclasses.replace(cp, use_tc_tiling_on_sc=False)
    dbuf = []
    for _ in range(2):
        dbuf += [
            pltpu.VMEM((chunk, E_DIM), jnp.float32),
            pltpu.VMEM((chunk, E_DIM), jnp.float32),
            pltpu.VMEM((chunk, R_DIM), jnp.float32),
            pltpu.VMEM((chunk, E_DIM * R_DIM), jnp.float32),
        ]
    run = pl.kernel(
        body,
        out_type=jax.ShapeDtypeStruct((batch,), jnp.float32),
        mesh=mesh,
        compiler_params=cp,
        scratch_types=[
            pltpu.VMEM((per_w,), jnp.int32),
            pltpu.VMEM((per_w,), jnp.int32),
            pltpu.VMEM((per_w,), jnp.int32),
        ] + dbuf + [
            pltpu.VMEM((per_w,), jnp.float32),
            pltpu.SemaphoreType.DMA,
            pltpu.SemaphoreType.DMA,
        ],
    )
    return run(head, relation, tail, entity_table, relation_table,
               transfer_table)


# speedup vs baseline: 1.5595x; 1.5122x over previous
import dataclasses
import functools

import jax
import jax.numpy as jnp
from jax import lax
from jax.experimental import pallas as pl
from jax.experimental.pallas import tpu as pltpu
from jax.experimental.pallas import tpu_sc as plsc

E_DIM = 64
R_DIM = 32
LANES = 16
NCHAIN = 4


def _score_kernel(per_w, chunk, nchunks,
                  head_hbm, rel_hbm, tail_hbm, ent_hbm, reltab_hbm, trans_hbm,
                  out_hbm, hidx, ridx, tidx,
                  hrows0, trows0, relemb0, mat0,
                  hrows1, trows1, relemb1, mat1,
                  outbuf, sem0, sem1):
    cid = lax.axis_index("core")
    sid = lax.axis_index("subcore")
    wid = sid * 2 + cid
    base = wid * per_w

    pltpu.sync_copy(head_hbm.at[pl.ds(base, per_w)], hidx)
    pltpu.sync_copy(tail_hbm.at[pl.ds(base, per_w)], tidx)
    pltpu.sync_copy(rel_hbm.at[pl.ds(base, per_w)], ridx)

    bufs = ((hrows0, trows0, relemb0, mat0, sem0),
            (hrows1, trows1, relemb1, mat1, sem1))

    def issue(ci, b):
        hr, tr, re, mt, sem = b
        off = ci * chunk
        hvec = hidx[pl.ds(off, chunk)]
        tvec = tidx[pl.ds(off, chunk)]
        rvec = ridx[pl.ds(off, chunk)]
        pltpu.async_copy(reltab_hbm.at[rvec], re, sem)
        pltpu.async_copy(trans_hbm.at[rvec], mt, sem)
        for i in range(chunk):
            pltpu.async_copy(ent_hbm.at[pl.ds(hvec[i], 1)],
                             hr.at[pl.ds(i, 1)], sem)
            pltpu.async_copy(ent_hbm.at[pl.ds(tvec[i], 1)],
                             tr.at[pl.ds(i, 1)], sem)

    def wait(b):
        hr, tr, re, mt, sem = b
        pltpu.make_async_copy(reltab_hbm.at[pl.ds(0, chunk)], re, sem).wait()
        pltpu.make_async_copy(trans_hbm.at[pl.ds(0, chunk)], mt, sem).wait()
        for i in range(chunk):
            pltpu.make_async_copy(ent_hbm.at[pl.ds(0, 1)],
                                  hr.at[pl.ds(i, 1)], sem).wait()
            pltpu.make_async_copy(ent_hbm.at[pl.ds(0, 1)],
                                  tr.at[pl.ds(i, 1)], sem).wait()

    def compute(ci, b):
        hr, tr, re, mt, _ = b
        off = ci * chunk

        def _sample(s, score_vec):
            zero = jnp.zeros((LANES,), jnp.float32)
            acc0 = [re[s, pl.ds(0, LANES)]] + [zero] * (NCHAIN - 1)
            acc1 = [re[s, pl.ds(LANES, LANES)]] + [zero] * (NCHAIN - 1)
            for kk in range(E_DIM // LANES):
                d = (hr[s, pl.ds(kk * LANES, LANES)]
                     - tr[s, pl.ds(kk * LANES, LANES)])
                for j in range(LANES):
                    k = kk * LANES + j
                    c = k % NCHAIN
                    dk = d[j]
                    acc0[c] = acc0[c] + dk * mt[s, pl.ds(k * R_DIM, LANES)]
                    acc1[c] = acc1[c] + dk * mt[s, pl.ds(k * R_DIM + LANES,
                                                         LANES)]
            t0 = (acc0[0] + acc0[1]) + (acc0[2] + acc0[3])
            t1 = (acc1[0] + acc1[1]) + (acc1[2] + acc1[3])
            score = jnp.sum(jnp.abs(t0) + jnp.abs(t1))
            lane = lax.iota(jnp.int32, LANES)
            return jnp.where(lane == s, score, score_vec)

        score_vec = lax.fori_loop(0, chunk, _sample,
                                  jnp.zeros((LANES,), jnp.float32))
        outbuf[pl.ds(off, chunk)] = score_vec

    issue(0, bufs[0])

    @pl.loop(0, nchunks, step=2)
    def _pair(ci):
        issue(ci + 1, bufs[1])
        wait(bufs[0])
        compute(ci, bufs[0])

        @pl.when(ci + 2 < nchunks)
        def _():
            issue(ci + 2, bufs[0])

        wait(bufs[1])
        compute(ci + 1, bufs[1])

    pltpu.sync_copy(outbuf, out_hbm.at[pl.ds(base, per_w)])


def kernel(head, relation, tail, entity_table, relation_table, transfer_table):
    batch = head.shape[0]
    num_workers = 32
    per_w = batch // num_workers
    chunk = LANES
    nchunks = per_w // chunk

    head = head.astype(jnp.int32)
    relation = relation.astype(jnp.int32)
    tail = tail.astype(jnp.int32)
    relation_pad = jnp.pad(relation_table, ((0, 0), (0, 128 - R_DIM)))

    mesh = plsc.VectorSubcoreMesh(core_axis_name="core", subcore_axis_name="subcore")
    body = functools.partial(_score_kernel, per_w, chunk, nchunks)
    cp = pltpu.CompilerParams()
    fields = pltpu.CompilerParams.__dataclass_fields__
    if "needs_layout_passes" in fields:
        cp = dataclasses.replace(cp, needs_layout_passes=False)
    if "use_tc_tiling_on_sc" in fields:
        cp = dataclasses.replace(cp, use_tc_tiling_on_sc=True)
    dbuf = []
    for _ in range(2):
        dbuf += [
            pltpu.VMEM((chunk, E_DIM), jnp.float32),
            pltpu.VMEM((chunk, E_DIM), jnp.float32),
            pltpu.VMEM((chunk, 128), jnp.float32),
            pltpu.VMEM((chunk, E_DIM * R_DIM), jnp.float32),
        ]
    run = pl.kernel(
        body,
        out_type=jax.ShapeDtypeStruct((batch,), jnp.float32),
        mesh=mesh,
        compiler_params=cp,
        scratch_types=[
            pltpu.VMEM((per_w,), jnp.int32),
            pltpu.VMEM((per_w,), jnp.int32),
            pltpu.VMEM((per_w,), jnp.int32),
        ] + dbuf + [
            pltpu.VMEM((per_w,), jnp.float32),
            pltpu.SemaphoreType.DMA,
            pltpu.SemaphoreType.DMA,
        ],
    )
    return run(head, relation, tail, entity_table, relation_pad,
               transfer_table)
